# baseline (speedup 1.0000x reference)
import functools

import jax
import jax.numpy as jnp
from jax import lax
from jax.experimental import pallas as pl
from jax.experimental.pallas import tpu as pltpu
from jax.experimental.pallas import tpu_sc as plsc

B = 16384
D = 64
OUT = 3
V = 1000000
TR_BLK = 32768

_info = plsc.get_sparse_core_info()
NC = _info.num_cores
NS = _info.num_subcores
NW = NC * NS
B_PER_W = B // NW

_mesh = plsc.VectorSubcoreMesh(core_axis_name="c", subcore_axis_name="s")


def _tr_body(*refs):
    t_refs, o_ref = refs[:8], refs[8]
    i = jnp.concatenate([r[...] for r in t_refs], axis=0)
    i = i.astype(jnp.bfloat16)
    qq = lax.broadcasted_iota(jnp.int32, (32, 128), 0)
    rr = lax.broadcasted_iota(jnp.int32, (32, 128), 1)
    sels = [
        jnp.where(rr == 4 * qq + m, jnp.float32(1), jnp.float32(0))
        .astype(jnp.bfloat16)
        for m in range(4)
    ]
    for k in range(TR_BLK // 128):
        blk = i[:, k * 128:(k + 1) * 128]
        for m in range(4):
            t = lax.dot_general(
                sels[m], blk, (((1,), (1,)), ((), ())),
                preferred_element_type=jnp.float32,
            )
            u = lax.bitcast_convert_type(t, jnp.uint32)
            hi = u & jnp.uint32(0xFFFF0000)
            lo = u >> 16
            o_ref[k * 32:(k + 1) * 32, m * 32:(m + 1) * 32] = (
                lax.bitcast_convert_type(
                    hi[:, :D // 2] | lo[:, D // 2:], jnp.float32
                )
            )


def _relayout(table_t):
    def _row_spec(j):
        return pl.BlockSpec((8, TR_BLK), lambda i, j=j: (j, i))

    return pl.pallas_call(
        _tr_body,
        grid=(pl.cdiv(V, TR_BLK),),
        in_specs=[_row_spec(j) for j in range(8)],
        out_specs=pl.BlockSpec((TR_BLK // 4, 128), lambda i: (i, 0)),
        out_shape=jax.ShapeDtypeStruct((V // 4, 128), jnp.float32),
    )(*([table_t] * 8))


@functools.partial(
    pl.kernel,
    mesh=_mesh,
    out_type=jax.ShapeDtypeStruct((B, 128), jnp.float32),
    scratch_types=[
        pltpu.VMEM((B_PER_W,), jnp.int32),
        pltpu.VMEM((B_PER_W, 128), jnp.float32),
        pltpu.SemaphoreType.DMA,
    ],
)
def _sc_gather(table_hbm, idx_hbm, out_hbm, idx_v, rows_v, sem):
    wid = lax.axis_index("s") * NC + lax.axis_index("c")
    base = wid * B_PER_W
    pltpu.sync_copy(idx_hbm.at[pl.ds(base, B_PER_W)], idx_v)

    def fire(g, carry):
        vg = idx_v[pl.ds(g * 16, 16)] >> 2
        for l in range(16):
            r = vg[l]
            pltpu.async_copy(
                table_hbm.at[pl.ds(r, 1)], rows_v.at[pl.ds(g * 16 + l, 1)], sem
            )
        return carry

    lax.fori_loop(0, B_PER_W // 16, fire, 0)

    def drain(j, carry):
        pltpu.make_async_copy(
            table_hbm.at[pl.ds(0, 1)], rows_v.at[pl.ds(j, 1)], sem
        ).wait()
        return carry

    lax.fori_loop(0, B_PER_W, drain, 0)

    pltpu.sync_copy(rows_v, out_hbm.at[pl.ds(base, B_PER_W)])


LOSS_BLK = 4096


def _tc_loss_body(emb_ref, x_ref, y_ref, w_ref, out_ref):
    u = lax.bitcast_convert_type(emb_ref[...], jnp.uint32)
    w = w_ref[...]
    m_vec = x_ref[...] & 3
    dn = (((1,), (1,)), ((), ()))
    yhat = jnp.zeros((LOSS_BLK, OUT), jnp.float32)
    for m in range(4):
        um = u[:, m * 32:(m + 1) * 32]
        a = lax.bitcast_convert_type(um & jnp.uint32(0xFFFF0000), jnp.float32)
        b = lax.bitcast_convert_type(um << 16, jnp.float32)
        ym = lax.dot_general(
            a, w[:, :D // 2], dn, preferred_element_type=jnp.float32
        ) + lax.dot_general(
            b, w[:, D // 2:], dn, preferred_element_type=jnp.float32
        )
        msk = jnp.where(m_vec == m, jnp.float32(1), jnp.float32(0))
        yhat = yhat + msk[:, None] * ym
    d = yhat - y_ref[...]
    partial = jnp.sum(d * d)

    @pl.when(pl.program_id(0) == 0)
    def _init():
        out_ref[0, 0] = 0.0

    out_ref[0, 0] += partial


def _tc_loss(emb, x, y_true, fc_weight):
    return pl.pallas_call(
        _tc_loss_body,
        grid=(B // LOSS_BLK,),
        in_specs=[
            pl.BlockSpec((LOSS_BLK, 128), lambda i: (i, 0)),
            pl.BlockSpec((LOSS_BLK,), lambda i: (i,)),
            pl.BlockSpec((LOSS_BLK, OUT), lambda i: (i, 0)),
            pl.BlockSpec((OUT, D), lambda i: (0, 0)),
        ],
        out_specs=pl.BlockSpec(memory_space=pltpu.SMEM),
        out_shape=jax.ShapeDtypeStruct((1, 1), jnp.float32),
    )(emb, x, y_true, fc_weight)


def kernel(x, y_true, embedding_table, fc_weight):
    table_t = embedding_table.T
    table_r = _relayout(table_t)
    emb = _sc_gather(table_r, x)
    loss = _tc_loss(emb, x, y_true, fc_weight)
    return loss[0, 0]


# speedup vs baseline: 1.2018x; 1.2018x over previous
import functools

import jax
import jax.numpy as jnp
from jax import lax
from jax.experimental import pallas as pl
from jax.experimental.pallas import tpu as pltpu
from jax.experimental.pallas import tpu_sc as plsc

B = 16384
D = 64
OUT = 3
V = 1000000
TR_BLK = 32768

_info = plsc.get_sparse_core_info()
NC = _info.num_cores
NS = _info.num_subcores
NW = NC * NS
B_PER_W = B // NW

_mesh = plsc.VectorSubcoreMesh(core_axis_name="c", subcore_axis_name="s")


def _tr_body(*refs):
    t_refs, o_ref = refs[:8], refs[8]
    i = jnp.concatenate([r[...] for r in t_refs], axis=0)
    i = i.astype(jnp.bfloat16)
    ii = lax.broadcasted_iota(jnp.int32, (128, 128), 0)
    jj = lax.broadcasted_iota(jnp.int32, (128, 128), 1)
    eye = jnp.where(ii == jj, jnp.float32(1), jnp.float32(0))
    eye = eye.astype(jnp.bfloat16)
    for k in range(TR_BLK // 128):
        blk = i[:, k * 128:(k + 1) * 128]
        t = lax.dot_general(
            eye, blk, (((1,), (1,)), ((), ())),
            preferred_element_type=jnp.float32,
        )
        u = lax.bitcast_convert_type(t, jnp.uint32)
        hi = u & jnp.uint32(0xFFFF0000)
        lo = u >> 16
        o_ref[k * 128:(k + 1) * 128, :] = lax.bitcast_convert_type(
            hi[:, :D // 2] | lo[:, D // 2:], jnp.float32
        )


def _relayout(table_t):
    def _row_spec(j):
        return pl.BlockSpec((8, TR_BLK), lambda i, j=j: (j, i))

    return pl.pallas_call(
        _tr_body,
        grid=(pl.cdiv(V, TR_BLK),),
        in_specs=[_row_spec(j) for j in range(8)],
        out_specs=pl.BlockSpec((TR_BLK, D // 2), lambda i: (i, 0)),
        out_shape=jax.ShapeDtypeStruct((V, D // 2), jnp.float32),
    )(*([table_t] * 8))


@functools.partial(
    pl.kernel,
    mesh=_mesh,
    out_type=jax.ShapeDtypeStruct((B, D // 2), jnp.float32),
    scratch_types=[
        pltpu.VMEM((B_PER_W,), jnp.int32),
        pltpu.VMEM((B_PER_W, D // 2), jnp.float32),
        pltpu.SemaphoreType.DMA,
    ],
)
def _sc_gather(table_hbm, idx_hbm, out_hbm, idx_v, rows_v, sem):
    wid = lax.axis_index("s") * NC + lax.axis_index("c")
    base = wid * B_PER_W
    pltpu.sync_copy(idx_hbm.at[pl.ds(base, B_PER_W)], idx_v)

    def fire(g, carry):
        vg = idx_v[pl.ds(g * 16, 16)]
        for l in range(16):
            r = vg[l]
            pltpu.async_copy(
                table_hbm.at[pl.ds(r, 1)], rows_v.at[pl.ds(g * 16 + l, 1)], sem
            )
        return carry

    lax.fori_loop(0, B_PER_W // 16, fire, 0)

    def drain(j, carry):
        pltpu.make_async_copy(
            table_hbm.at[pl.ds(0, 1)], rows_v.at[pl.ds(j, 1)], sem
        ).wait()
        return carry

    lax.fori_loop(0, B_PER_W, drain, 0)

    pltpu.sync_copy(rows_v, out_hbm.at[pl.ds(base, B_PER_W)])


LOSS_BLK = 4096


def _tc_loss_body(emb_ref, y_ref, w_ref, out_ref):
    u = lax.bitcast_convert_type(emb_ref[...], jnp.uint32)
    a = lax.bitcast_convert_type(u & jnp.uint32(0xFFFF0000), jnp.float32)
    b = lax.bitcast_convert_type(u << 16, jnp.float32)
    w = w_ref[...]
    dn = (((1,), (1,)), ((), ()))
    yhat = lax.dot_general(
        a, w[:, :D // 2], dn, preferred_element_type=jnp.float32
    ) + lax.dot_general(
        b, w[:, D // 2:], dn, preferred_element_type=jnp.float32
    )
    d = yhat - y_ref[...]
    partial = jnp.sum(d * d)

    @pl.when(pl.program_id(0) == 0)
    def _init():
        out_ref[0, 0] = 0.0

    out_ref[0, 0] += partial


def _tc_loss(emb, y_true, fc_weight):
    return pl.pallas_call(
        _tc_loss_body,
        grid=(B // LOSS_BLK,),
        in_specs=[
            pl.BlockSpec((LOSS_BLK, D // 2), lambda i: (i, 0)),
            pl.BlockSpec((LOSS_BLK, OUT), lambda i: (i, 0)),
            pl.BlockSpec((OUT, D), lambda i: (0, 0)),
        ],
        out_specs=pl.BlockSpec(memory_space=pltpu.SMEM),
        out_shape=jax.ShapeDtypeStruct((1, 1), jnp.float32),
    )(emb, y_true, fc_weight)


def kernel(x, y_true, embedding_table, fc_weight):
    table_t = embedding_table.T
    table_r = _relayout(table_t)
    emb = _sc_gather(table_r, x)
    loss = _tc_loss(emb, y_true, fc_weight)
    return loss[0, 0]
